# Initial kernel scaffold; baseline (speedup 1.0000x reference)
import jax
import jax.numpy as jnp
from jax.experimental import pallas as pl


def kernel(role_ids, namespace_ids, label_ids, path_ids, depth_ids, position_ids, numeric_features, role_table, namespace_table, label_table, path_table, depth_table, position_table, num_w, num_b):
    raise NotImplementedError("write your pallas kernel here")


# speedup vs baseline: 3.1217x; 3.1217x over previous
import functools

import jax
import jax.numpy as jnp
from jax import lax
from jax.experimental import pallas as pl
from jax.experimental.pallas import tpu as pltpu
from jax.experimental.pallas import tpu_sc as plsc

D = 64
N_TABLES = 6


def _sc_gather_sum(n_tokens, chunk, table_shapes):
    info = plsc.get_sparse_core_info()
    nc, ns = info.num_cores, info.num_subcores
    nw = nc * ns
    assert n_tokens % (nw * chunk) == 0
    n_per_w = n_tokens // nw
    n_chunks = n_per_w // chunk

    mesh = plsc.VectorSubcoreMesh(core_axis_name="c", subcore_axis_name="s")

    scratch = (
        [pltpu.VMEM((chunk,), jnp.int32) for _ in range(N_TABLES)]
        + [pltpu.VMEM((chunk, 2 * D), jnp.float32) for _ in range(N_TABLES)]
        + [pltpu.VMEM((chunk, D), jnp.float32), pltpu.SemaphoreType.DMA]
    )

    @functools.partial(
        pl.kernel,
        mesh=mesh,
        out_type=jax.ShapeDtypeStruct((n_tokens, D), jnp.float32),
        scratch_types=scratch,
    )
    def k(idx_hbm, t0, t1, t2, t3, t4, t5, out_hbm,
          i0, i1, i2, i3, i4, i5, r0, r1, r2, r3, r4, r5, acc, sem):
        tables = (t0, t1, t2, t3, t4, t5)
        idx_bufs = (i0, i1, i2, i3, i4, i5)
        rows = (r0, r1, r2, r3, r4, r5)
        wid = lax.axis_index("s") * nc + lax.axis_index("c")
        base = wid * n_per_w

        def chunk_body(c, carry):
            start = base + c * chunk
            for t in range(N_TABLES):
                pltpu.sync_copy(idx_hbm.at[t, pl.ds(start, chunk)], idx_bufs[t])
            cps = [pltpu.async_copy(tables[t].at[idx_bufs[t]], rows[t], sem)
                   for t in range(N_TABLES)]
            for cp in cps:
                cp.wait()

            def tok_body(t, carry2):
                for dch in range(D // 16):
                    sl = pl.ds(dch * 16, 16)
                    v = rows[0][t, sl]
                    for k2 in range(1, N_TABLES):
                        v = v + rows[k2][t, sl]
                    acc[t, sl] = v
                return carry2

            lax.fori_loop(0, chunk, tok_body, 0)
            pltpu.sync_copy(acc, out_hbm.at[pl.ds(start, chunk)])
            return carry

        lax.fori_loop(0, n_chunks, chunk_body, 0)

    return k


def _tc_finish_body(emb_ref, nf_ref, w_ref, b_ref, out_ref):
    out_ref[...] = (
        emb_ref[...]
        + jnp.dot(nf_ref[...], w_ref[...], preferred_element_type=jnp.float32)
        + b_ref[...]
    )


def kernel(role_ids, namespace_ids, label_ids, path_ids, depth_ids, position_ids,
           numeric_features, role_table, namespace_table, label_table, path_table,
           depth_table, position_table, num_w, num_b):
    B, T = role_ids.shape
    n = B * T
    nf = numeric_features.shape[-1]

    idx_all = jnp.stack([
        role_ids.reshape(n), namespace_ids.reshape(n), label_ids.reshape(n),
        path_ids.reshape(n), depth_ids.reshape(n), position_ids.reshape(n),
    ]).astype(jnp.int32)

    tables = (role_table, namespace_table, label_table, path_table,
              depth_table, position_table)

    tables = tuple(jnp.pad(t, ((0, 0), (0, D))) for t in tables)

    sc = _sc_gather_sum(n, 128, tuple(t.shape for t in tables))
    emb = sc(idx_all, *tables)

    blk = 2048
    out = pl.pallas_call(
        _tc_finish_body,
        grid=(n // blk,),
        in_specs=[
            pl.BlockSpec((blk, D), lambda i: (i, 0)),
            pl.BlockSpec((blk, nf), lambda i: (i, 0)),
            pl.BlockSpec((nf, D), lambda i: (0, 0)),
            pl.BlockSpec((1, D), lambda i: (0, 0)),
        ],
        out_specs=pl.BlockSpec((blk, D), lambda i: (i, 0)),
        out_shape=jax.ShapeDtypeStruct((n, D), jnp.float32),
    )(emb, numeric_features.reshape(n, nf), num_w, num_b.reshape(1, D))

    return out.reshape(B, T, D)
